# baseline (speedup 1.0000x reference)
import functools

import jax
import jax.numpy as jnp
from jax import lax
from jax.experimental import pallas as pl
from jax.experimental.pallas import tpu as pltpu
from jax.experimental.pallas import tpu_sc as plsc

_NUM_CORES = 2
_NUM_SUBCORES = 16
_NUM_WORKERS = _NUM_CORES * _NUM_SUBCORES
_LANES = 16


@functools.partial(jax.jit, static_argnums=(1, 2, 3))
def _position_embedding(pattern, b, h, w):
    nch = pattern.shape[0]
    d = nch // 2
    cpw = nch // _NUM_WORKERS
    assert d % cpw == 0 and w % _LANES == 0

    mesh = plsc.VectorSubcoreMesh(core_axis_name="c", subcore_axis_name="s")

    @functools.partial(
        pl.kernel,
        mesh=mesh,
        out_type=jax.ShapeDtypeStruct((b, nch, h, w), jnp.float32),
        scratch_types=[
            pltpu.VMEM((cpw, w), jnp.float32),
            pltpu.VMEM((cpw, h, w), jnp.float32),
            pltpu.SemaphoreType.DMA,
        ],
    )
    def sc_kernel(pattern_hbm, out_hbm, pat_v, buf_v, sem):
        wid = lax.axis_index("s") * _NUM_CORES + lax.axis_index("c")
        c0 = wid * cpw
        pltpu.sync_copy(pattern_hbm.at[pl.ds(c0, cpw)], pat_v)

        @pl.when(wid < d // cpw)
        def _col_channels():
            for i in range(cpw):
                for xv in range(w // _LANES):
                    g = pat_v[i, pl.ds(xv * _LANES, _LANES)]
                    for y in range(h):
                        buf_v[i, y, pl.ds(xv * _LANES, _LANES)] = g

        @pl.when(wid >= d // cpw)
        def _row_channels():
            for i in range(cpw):
                for yv in range(h // _LANES):
                    g = pat_v[i, pl.ds(yv * _LANES, _LANES)]
                    for yl in range(_LANES):
                        val = jnp.full((_LANES,), g[yl], dtype=jnp.float32)
                        for xv in range(w // _LANES):
                            buf_v[i, yv * _LANES + yl,
                                  pl.ds(xv * _LANES, _LANES)] = val

        copies = [
            pltpu.make_async_copy(buf_v, out_hbm.at[bi, pl.ds(c0, cpw)], sem)
            for bi in range(b)
        ]
        for cp in copies:
            cp.start()
        for cp in copies:
            cp.wait()

    return sc_kernel(pattern)


def kernel(x, row_embed, col_embed):
    b = x.shape[0]
    h, w = x.shape[-2], x.shape[-1]
    pattern = jnp.concatenate([col_embed[:w].T, row_embed[:h].T], axis=0)
    return _position_embedding(pattern, b, h, w)


# speedup vs baseline: 4.3715x; 4.3715x over previous
import functools

import jax
import jax.numpy as jnp
from jax import lax
from jax.experimental import pallas as pl
from jax.experimental.pallas import tpu as pltpu
from jax.experimental.pallas import tpu_sc as plsc

_NUM_CORES = 2
_NUM_SUBCORES = 16
_NUM_WORKERS = _NUM_CORES * _NUM_SUBCORES
_LANES = 16


@functools.partial(jax.jit, static_argnums=(2, 3, 4))
def _position_embedding(row_embed, col_embed, b, h, w):
    d = row_embed.shape[-1]
    nch = 2 * d
    assert h == _NUM_WORKERS and d % _LANES == 0

    mesh = plsc.VectorSubcoreMesh(core_axis_name="c", subcore_axis_name="s")

    @functools.partial(
        pl.kernel,
        mesh=mesh,
        out_type=jax.ShapeDtypeStruct((b, h, w, nch), jnp.float32),
        scratch_types=[
            pltpu.VMEM((w, nch), jnp.float32),
            pltpu.VMEM((1, d), jnp.float32),
            pltpu.SemaphoreType.DMA,
        ],
    )
    def sc_kernel(row_hbm, col_hbm, out_hbm, slab_v, row_v, sem):
        y = lax.axis_index("s") * _NUM_CORES + lax.axis_index("c")
        pltpu.sync_copy(col_hbm.at[pl.ds(0, w)], slab_v.at[:, pl.ds(0, d)])
        pltpu.sync_copy(row_hbm.at[pl.ds(y, 1)], row_v)
        for j in range(d // _LANES):
            g = row_v[0, pl.ds(j * _LANES, _LANES)]
            for xi in range(w):
                slab_v[xi, pl.ds(d + j * _LANES, _LANES)] = g
        copies = [
            pltpu.make_async_copy(slab_v, out_hbm.at[bi, y], sem)
            for bi in range(b)
        ]
        for cp in copies:
            cp.start()
        for cp in copies:
            cp.wait()

    out = sc_kernel(row_embed, col_embed)
    return jnp.transpose(out, (0, 3, 1, 2))


def kernel(x, row_embed, col_embed):
    b = x.shape[0]
    h, w = x.shape[-2], x.shape[-1]
    return _position_embedding(row_embed, col_embed, b, h, w)


# speedup vs baseline: 4.4385x; 1.0153x over previous
import functools

import jax
import jax.numpy as jnp
from jax import lax
from jax.experimental import pallas as pl
from jax.experimental.pallas import tpu as pltpu
from jax.experimental.pallas import tpu_sc as plsc

_NUM_CORES = 2
_NUM_SUBCORES = 16
_NUM_WORKERS = _NUM_CORES * _NUM_SUBCORES
_LANES = 16


@functools.partial(jax.jit, static_argnums=(2, 3, 4))
def _position_embedding(row_embed, col_embed, b, h, w):
    d = row_embed.shape[-1]
    nch = 2 * d
    assert h == _NUM_WORKERS and d % _LANES == 0

    mesh = plsc.VectorSubcoreMesh(core_axis_name="c", subcore_axis_name="s")

    @functools.partial(
        pl.kernel,
        mesh=mesh,
        out_type=jax.ShapeDtypeStruct((b, h, w, nch), jnp.float32),
        scratch_types=[
            pltpu.VMEM((w, nch), jnp.float32),
            pltpu.VMEM((1, d), jnp.float32),
            pltpu.SemaphoreType.DMA,
        ],
    )
    def sc_kernel(row_hbm, col_hbm, out_hbm, slab_v, row_v, sem):
        y = lax.axis_index("s") * _NUM_CORES + lax.axis_index("c")
        col_cp = pltpu.make_async_copy(
            col_hbm.at[pl.ds(0, w)], slab_v.at[:, pl.ds(0, d)], sem)
        row_cp = pltpu.make_async_copy(row_hbm.at[pl.ds(y, 1)], row_v, sem)
        col_cp.start()
        row_cp.start()
        col_cp.wait()
        row_cp.wait()

        gs = [row_v[0, pl.ds(j * _LANES, _LANES)] for j in range(d // _LANES)]

        def _fill(xi, carry):
            for j, g in enumerate(gs):
                slab_v[xi, pl.ds(d + j * _LANES, _LANES)] = g
            return carry

        lax.fori_loop(0, w, _fill, 0)

        def _fire(bi, carry):
            pltpu.make_async_copy(slab_v, out_hbm.at[bi, y], sem).start()
            return carry

        def _drain(bi, carry):
            pltpu.make_async_copy(slab_v, out_hbm.at[bi, y], sem).wait()
            return carry

        lax.fori_loop(0, b, _fire, 0)
        lax.fori_loop(0, b, _drain, 0)

    out = sc_kernel(row_embed, col_embed)
    return jnp.transpose(out, (0, 3, 1, 2))


def kernel(x, row_embed, col_embed):
    b = x.shape[0]
    h, w = x.shape[-2], x.shape[-1]
    return _position_embedding(row_embed, col_embed, b, h, w)
